# baseline (speedup 1.0000x reference)
import functools

import jax
import jax.numpy as jnp
from jax.experimental import pallas as pl
from jax.experimental.pallas import tpu as pltpu


def _row_tile(n, target=512):
    if n <= target:
        return n
    t = (target // 8) * 8
    while t >= 8:
        if n % t == 0:
            return t
        t -= 8
    return n


def _params(sems):
    return pltpu.CompilerParams(dimension_semantics=sems,
                                vmem_limit_bytes=56 * 1024 * 1024)


def _s0_kernel(x_ref, w0_ref, s0_ref):
    xb = x_ref[...].astype(jnp.bfloat16)
    s0_ref[...] = jnp.dot(xb, w0_ref[...],
                          preferred_element_type=jnp.float32).astype(jnp.bfloat16)


def _stage1(x, w0b):
    n, din = x.shape
    dout = w0b.shape[1]
    tm = _row_tile(n, 1024)
    return pl.pallas_call(
        _s0_kernel,
        out_shape=jax.ShapeDtypeStruct((n, dout), jnp.bfloat16),
        grid_spec=pltpu.PrefetchScalarGridSpec(
            num_scalar_prefetch=0,
            grid=(n // tm,),
            in_specs=[
                pl.BlockSpec((tm, din), lambda i: (i, 0)),
                pl.BlockSpec((din, dout), lambda i: (0, 0)),
            ],
            out_specs=pl.BlockSpec((tm, dout), lambda i: (i, 0)),
        ),
        compiler_params=_params(("parallel",)),
    )(x, w0b)


_G_SCALE = 256.0
_S_SCALE = 16.0
_F8 = jnp.float8_e4m3fn
_F8_MAX = 448.0


def _layer0_kernel(g_ref, s0_ref, b0_ref, w1_ref, h0_ref, s1_ref, g8_ref):
    gf = g_ref[...]
    gb = gf.astype(jnp.bfloat16)
    g8_ref[...] = (gf * _G_SCALE).astype(_F8)
    acc = jnp.dot(gb, s0_ref[...], preferred_element_type=jnp.float32)
    h = jnp.tanh(acc + b0_ref[...])
    hb = h.astype(jnp.bfloat16)
    h0_ref[...] = hb
    s1 = jnp.dot(hb, w1_ref[...], preferred_element_type=jnp.float32)
    s1_ref[...] = jnp.clip(s1 * _S_SCALE, -_F8_MAX, _F8_MAX).astype(_F8)


def _stage2(g, s0, b0, w1b, tm):
    n, k = g.shape
    d = s0.shape[1]
    return pl.pallas_call(
        _layer0_kernel,
        out_shape=(jax.ShapeDtypeStruct((n, d), jnp.bfloat16),
                   jax.ShapeDtypeStruct((n, d), _F8),
                   jax.ShapeDtypeStruct((n, k), _F8)),
        grid_spec=pltpu.PrefetchScalarGridSpec(
            num_scalar_prefetch=0,
            grid=(n // tm,),
            in_specs=[
                pl.BlockSpec((tm, k), lambda i: (i, 0)),
                pl.BlockSpec((k, d), lambda i: (0, 0)),
                pl.BlockSpec((1, d), lambda i: (0, 0)),
                pl.BlockSpec((d, d), lambda i: (0, 0)),
            ],
            out_specs=(pl.BlockSpec((tm, d), lambda i: (i, 0)),
                       pl.BlockSpec((tm, d), lambda i: (i, 0)),
                       pl.BlockSpec((tm, k), lambda i: (i, 0))),
        ),
        compiler_params=_params(("parallel",)),
    )(g, s0, b0, w1b)


def _layer1_kernel(alpha_ref, g_ref, s1_ref, b1_ref, h0_ref, x_ref, rw_ref,
                   gamma_ref, nb_ref, o_ref, *, eps):
    acc = jnp.dot(g_ref[...], s1_ref[...], preferred_element_type=jnp.float32)
    acc = acc * (1.0 / (_G_SCALE * _S_SCALE))
    h1 = jnp.tanh(acc + b1_ref[...])
    res = jnp.dot(x_ref[...], rw_ref[...], preferred_element_type=jnp.float32)
    c = (alpha_ref[0] * h0_ref[...].astype(jnp.float32)
         + alpha_ref[1] * h1 + res)
    d = c.shape[-1]
    mean = jnp.mean(c, axis=-1, keepdims=True)
    cc = c - mean
    var = jnp.sum(cc * cc, axis=-1, keepdims=True) / (d - 1)
    denom = jnp.sqrt(var) + eps
    inv = pl.reciprocal(denom, approx=True)
    inv = inv * (2.0 - denom * inv)
    o_ref[...] = (gamma_ref[...] * (cc * inv) + nb_ref[...]).astype(o_ref.dtype)


def _stage3(g8, s1, b1, h0, x, rw, alpha2, gamma, nbias, tm, eps):
    n, k = g8.shape
    d = s1.shape[1]
    din = x.shape[1]
    row = lambda i: (i, 0)
    const = lambda i: (0, 0)
    return pl.pallas_call(
        functools.partial(_layer1_kernel, eps=eps),
        out_shape=jax.ShapeDtypeStruct((n, d), x.dtype),
        grid_spec=pltpu.PrefetchScalarGridSpec(
            num_scalar_prefetch=0,
            grid=(n // tm,),
            in_specs=[
                pl.BlockSpec(memory_space=pltpu.MemorySpace.SMEM),
                pl.BlockSpec((tm, k), row),
                pl.BlockSpec((k, d), const),
                pl.BlockSpec((1, d), const),
                pl.BlockSpec((tm, d), row),
                pl.BlockSpec((tm, din), row),
                pl.BlockSpec((din, d), const),
                pl.BlockSpec((tm, d), row),
                pl.BlockSpec((tm, d), row),
            ],
            out_specs=pl.BlockSpec((tm, d), row),
        ),
        compiler_params=_params(("parallel",)),
    )(alpha2, g8, s1, b1, h0, x, rw, gamma, nbias)


def kernel(g, x, w0, b0, w1, b1, alpha, res_weight, gamma, norm_bias):
    n = g.shape[0]
    d = w0.shape[1]
    eps = 1e-10
    tm2 = _row_tile(n, 1024)
    tm3 = _row_tile(n, 512)

    w0b = w0.astype(jnp.bfloat16)
    w1b = w1.astype(jnp.bfloat16)
    b0r = b0.reshape(1, d)
    b1r = b1.reshape(1, d)
    alpha2 = alpha.reshape(-1).astype(jnp.float32)

    s0 = _stage1(x, w0b)
    h0, s1, g8 = _stage2(g, s0, b0r, w1b, tm2)
    return _stage3(g8, s1, b1r, h0, x, res_weight, alpha2, gamma, norm_bias,
                   tm3, eps)


# speedup vs baseline: 1.0082x; 1.0082x over previous
import functools

import jax
import jax.numpy as jnp
from jax.experimental import pallas as pl
from jax.experimental.pallas import tpu as pltpu


def _row_tile(n, target=512):
    if n <= target:
        return n
    t = (target // 8) * 8
    while t >= 8:
        if n % t == 0:
            return t
        t -= 8
    return n


def _params(sems):
    return pltpu.CompilerParams(dimension_semantics=sems,
                                vmem_limit_bytes=56 * 1024 * 1024)


def _s0_kernel(x_ref, w0_ref, s0_ref):
    xb = x_ref[...].astype(jnp.bfloat16)
    s0_ref[...] = jnp.dot(xb, w0_ref[...],
                          preferred_element_type=jnp.float32).astype(jnp.bfloat16)


def _stage1(x, w0b):
    n, din = x.shape
    dout = w0b.shape[1]
    tm = _row_tile(n, 1024)
    return pl.pallas_call(
        _s0_kernel,
        out_shape=jax.ShapeDtypeStruct((n, dout), jnp.bfloat16),
        grid_spec=pltpu.PrefetchScalarGridSpec(
            num_scalar_prefetch=0,
            grid=(n // tm,),
            in_specs=[
                pl.BlockSpec((tm, din), lambda i: (i, 0)),
                pl.BlockSpec((din, dout), lambda i: (0, 0)),
            ],
            out_specs=pl.BlockSpec((tm, dout), lambda i: (i, 0)),
        ),
        compiler_params=_params(("parallel",)),
    )(x, w0b)


_G_SCALE = 256.0
_S_SCALE = 16.0
_F8 = jnp.float8_e4m3fn
_F8_MAX = 448.0


def _layer0_kernel(g_ref, s0_ref, b0_ref, w1_ref, h0_ref, s1_ref, g8_ref):
    gf = g_ref[...]
    gb = gf.astype(jnp.bfloat16)
    g8_ref[...] = (gf * _G_SCALE).astype(_F8)
    acc = jnp.dot(gb, s0_ref[...], preferred_element_type=jnp.float32)
    h = jnp.tanh(acc + b0_ref[...])
    hb = h.astype(jnp.bfloat16)
    h0_ref[...] = hb
    s1 = jnp.dot(hb, w1_ref[...], preferred_element_type=jnp.float32)
    s1_ref[...] = jnp.clip(s1 * _S_SCALE, -_F8_MAX, _F8_MAX).astype(_F8)


def _stage2(g, s0, b0, w1b, tm):
    n, k = g.shape
    d = s0.shape[1]
    return pl.pallas_call(
        _layer0_kernel,
        out_shape=(jax.ShapeDtypeStruct((n, d), jnp.bfloat16),
                   jax.ShapeDtypeStruct((n, d), _F8),
                   jax.ShapeDtypeStruct((n, k), _F8)),
        grid_spec=pltpu.PrefetchScalarGridSpec(
            num_scalar_prefetch=0,
            grid=(n // tm,),
            in_specs=[
                pl.BlockSpec((tm, k), lambda i: (i, 0)),
                pl.BlockSpec((k, d), lambda i: (0, 0)),
                pl.BlockSpec((1, d), lambda i: (0, 0)),
                pl.BlockSpec((d, d), lambda i: (0, 0)),
            ],
            out_specs=(pl.BlockSpec((tm, d), lambda i: (i, 0)),
                       pl.BlockSpec((tm, d), lambda i: (i, 0)),
                       pl.BlockSpec((tm, k), lambda i: (i, 0))),
        ),
        compiler_params=_params(("parallel",)),
    )(g, s0, b0, w1b)


def _layer1_kernel(alpha_ref, g_ref, s1_ref, b1_ref, h0_ref, x_ref, rw_ref,
                   gamma_ref, nb_ref, o_ref, *, eps):
    acc = jnp.dot(g_ref[...], s1_ref[...], preferred_element_type=jnp.float32)
    acc = acc * (1.0 / (_G_SCALE * _S_SCALE))
    h1 = jnp.tanh(acc + b1_ref[...])
    res = jnp.dot(x_ref[...], rw_ref[...], preferred_element_type=jnp.float32)
    c = (alpha_ref[0] * h0_ref[...].astype(jnp.float32)
         + alpha_ref[1] * h1 + res)
    d = c.shape[-1]
    mean = jnp.mean(c, axis=-1, keepdims=True)
    cc = c - mean
    var = jnp.sum(cc * cc, axis=-1, keepdims=True) / (d - 1)
    denom = jnp.sqrt(var) + eps
    inv = pl.reciprocal(denom, approx=True)
    inv = inv * (2.0 - denom * inv)
    o_ref[...] = (gamma_ref[...] * (cc * inv) + nb_ref[...]).astype(o_ref.dtype)


def _stage3(g8, s1, b1, h0, x, rw, alpha2, gamma, nbias, tm, eps):
    n, k = g8.shape
    d = s1.shape[1]
    din = x.shape[1]
    row = lambda i: (i, 0)
    const = lambda i: (0, 0)
    return pl.pallas_call(
        functools.partial(_layer1_kernel, eps=eps),
        out_shape=jax.ShapeDtypeStruct((n, d), x.dtype),
        grid_spec=pltpu.PrefetchScalarGridSpec(
            num_scalar_prefetch=0,
            grid=(n // tm,),
            in_specs=[
                pl.BlockSpec(memory_space=pltpu.MemorySpace.SMEM),
                pl.BlockSpec((tm, k), row),
                pl.BlockSpec((k, d), const),
                pl.BlockSpec((1, d), const),
                pl.BlockSpec((tm, d), row),
                pl.BlockSpec((tm, din), row),
                pl.BlockSpec((din, d), const),
                pl.BlockSpec((tm, d), row),
                pl.BlockSpec((tm, d), row),
            ],
            out_specs=pl.BlockSpec((tm, d), row),
        ),
        compiler_params=_params(("parallel",)),
    )(alpha2, g8, s1, b1, h0, x, rw, gamma, nbias)


def kernel(g, x, w0, b0, w1, b1, alpha, res_weight, gamma, norm_bias):
    n = g.shape[0]
    d = w0.shape[1]
    eps = 1e-10
    tm2 = _row_tile(n, 1024)
    tm3 = _row_tile(n, 1024)

    w0b = w0.astype(jnp.bfloat16)
    w1b = w1.astype(jnp.bfloat16)
    b0r = b0.reshape(1, d)
    b1r = b1.reshape(1, d)
    alpha2 = alpha.reshape(-1).astype(jnp.float32)

    s0 = _stage1(x, w0b)
    h0, s1, g8 = _stage2(g, s0, b0r, w1b, tm2)
    return _stage3(g8, s1, b1r, h0, x, res_weight, alpha2, gamma, norm_bias,
                   tm3, eps)


# speedup vs baseline: 1.1463x; 1.1370x over previous
import functools

import jax
import jax.numpy as jnp
from jax.experimental import pallas as pl
from jax.experimental.pallas import tpu as pltpu


def _row_tile(n, target=512):
    if n <= target:
        return n
    t = (target // 8) * 8
    while t >= 8:
        if n % t == 0:
            return t
        t -= 8
    return n


def _params(sems):
    return pltpu.CompilerParams(dimension_semantics=sems,
                                vmem_limit_bytes=48 * 1024 * 1024)


def _s0_kernel(x_ref, w0_ref, s0_ref):
    xb = x_ref[...].astype(jnp.bfloat16)
    s0_ref[...] = jnp.dot(xb, w0_ref[...],
                          preferred_element_type=jnp.float32).astype(jnp.bfloat16)


def _stage1(x, w0b):
    n, din = x.shape
    dout = w0b.shape[1]
    tm = _row_tile(n, 1024)
    return pl.pallas_call(
        _s0_kernel,
        out_shape=jax.ShapeDtypeStruct((n, dout), jnp.bfloat16),
        grid_spec=pltpu.PrefetchScalarGridSpec(
            num_scalar_prefetch=0,
            grid=(n // tm,),
            in_specs=[
                pl.BlockSpec((tm, din), lambda i: (i, 0)),
                pl.BlockSpec((din, dout), lambda i: (0, 0)),
            ],
            out_specs=pl.BlockSpec((tm, dout), lambda i: (i, 0)),
        ),
        compiler_params=_params(("parallel",)),
    )(x, w0b)


_G_SCALE = 256.0
_S_SCALE = 16.0
_F8 = jnp.float8_e4m3fn
_F8_MAX = 448.0


def _layer0_kernel(g_ref, s0_ref, b0_ref, w1_ref, h0_ref, s1_ref, g8_ref):
    gf = g_ref[...]
    gb = gf.astype(jnp.bfloat16)
    g8_ref[...] = (gf * _G_SCALE).astype(_F8)
    acc = jnp.dot(gb, s0_ref[...], preferred_element_type=jnp.float32)
    h = jnp.tanh(acc + b0_ref[...])
    hb = h.astype(jnp.bfloat16)
    h0_ref[...] = hb
    s1 = jnp.dot(hb, w1_ref[...], preferred_element_type=jnp.float32)
    s1_ref[...] = jnp.clip(s1 * _S_SCALE, -_F8_MAX, _F8_MAX).astype(_F8)


def _stage2(g, s0, b0, w1b, tm):
    n, k = g.shape
    d = s0.shape[1]
    return pl.pallas_call(
        _layer0_kernel,
        out_shape=(jax.ShapeDtypeStruct((n, d), jnp.bfloat16),
                   jax.ShapeDtypeStruct((n, d), _F8),
                   jax.ShapeDtypeStruct((n, k), _F8)),
        grid_spec=pltpu.PrefetchScalarGridSpec(
            num_scalar_prefetch=0,
            grid=(n // tm,),
            in_specs=[
                pl.BlockSpec((tm, k), lambda i: (i, 0)),
                pl.BlockSpec((k, d), lambda i: (0, 0)),
                pl.BlockSpec((1, d), lambda i: (0, 0)),
                pl.BlockSpec((d, d), lambda i: (0, 0)),
            ],
            out_specs=(pl.BlockSpec((tm, d), lambda i: (i, 0)),
                       pl.BlockSpec((tm, d), lambda i: (i, 0)),
                       pl.BlockSpec((tm, k), lambda i: (i, 0))),
        ),
        compiler_params=_params(("parallel",)),
    )(g, s0, b0, w1b)


def _layer1_kernel(alpha_ref, g_ref, s1_ref, b1_ref, h0_ref, x_ref, rw_ref,
                   gamma_ref, nb_ref, o_ref, *, eps):
    acc = jnp.dot(g_ref[...], s1_ref[...], preferred_element_type=jnp.float32)
    acc = acc * (1.0 / (_G_SCALE * _S_SCALE))
    h1 = jnp.tanh(acc + b1_ref[...])
    res = jnp.dot(x_ref[...], rw_ref[...], preferred_element_type=jnp.float32)
    c = (alpha_ref[0] * h0_ref[...].astype(jnp.float32)
         + alpha_ref[1] * h1 + res)
    d = c.shape[-1]
    mean = jnp.mean(c, axis=-1, keepdims=True)
    cc = c - mean
    var = jnp.sum(cc * cc, axis=-1, keepdims=True) / (d - 1)
    denom = jnp.sqrt(var) + eps
    inv = pl.reciprocal(denom, approx=True)
    inv = inv * (2.0 - denom * inv)
    o_ref[...] = (gamma_ref[...] * (cc * inv) + nb_ref[...]).astype(o_ref.dtype)


def _stage3(g8, s1, b1, h0, x, rw, alpha2, gamma, nbias, tm, eps):
    n, k = g8.shape
    d = s1.shape[1]
    din = x.shape[1]
    row = lambda i: (i, 0)
    const = lambda i: (0, 0)
    return pl.pallas_call(
        functools.partial(_layer1_kernel, eps=eps),
        out_shape=jax.ShapeDtypeStruct((n, d), x.dtype),
        grid_spec=pltpu.PrefetchScalarGridSpec(
            num_scalar_prefetch=0,
            grid=(n // tm,),
            in_specs=[
                pl.BlockSpec(memory_space=pltpu.MemorySpace.SMEM),
                pl.BlockSpec((tm, k), row),
                pl.BlockSpec((k, d), const),
                pl.BlockSpec((1, d), const),
                pl.BlockSpec((tm, d), row),
                pl.BlockSpec((tm, din), row),
                pl.BlockSpec((din, d), const),
                pl.BlockSpec((tm, d), row),
                pl.BlockSpec((tm, d), row),
            ],
            out_specs=pl.BlockSpec((tm, d), row),
        ),
        compiler_params=_params(("parallel",)),
    )(alpha2, g8, s1, b1, h0, x, rw, gamma, nbias)


def kernel(g, x, w0, b0, w1, b1, alpha, res_weight, gamma, norm_bias):
    n = g.shape[0]
    d = w0.shape[1]
    eps = 1e-10
    tm2 = _row_tile(n, 1024)
    tm3 = _row_tile(n, 1024)

    w0b = w0.astype(jnp.bfloat16)
    w1b = w1.astype(jnp.bfloat16)
    b0r = b0.reshape(1, d)
    b1r = b1.reshape(1, d)
    alpha2 = alpha.reshape(-1).astype(jnp.float32)

    s0 = _stage1(x, w0b)
    h0, s1, g8 = _stage2(g, s0, b0r, w1b, tm2)
    return _stage3(g8, s1, b1r, h0, x, res_weight, alpha2, gamma, norm_bias,
                   tm3, eps)


# speedup vs baseline: 1.1810x; 1.0302x over previous
import functools

import jax
import jax.numpy as jnp
from jax.experimental import pallas as pl
from jax.experimental.pallas import tpu as pltpu


def _row_tile(n, target=512):
    if n <= target:
        return n
    t = (target // 8) * 8
    while t >= 8:
        if n % t == 0:
            return t
        t -= 8
    return n


def _params(sems):
    return pltpu.CompilerParams(dimension_semantics=sems,
                                vmem_limit_bytes=48 * 1024 * 1024)


_G_SCALE = 256.0
_S_SCALE = 16.0
_F8 = jnp.float8_e4m3fn
_F8_MAX = 448.0


def _layer0_kernel(g_ref, x_ref, w0_ref, b0_ref, w1_ref, h0_ref, s1_ref,
                   g8_ref):
    gf = g_ref[...]
    gb = gf.astype(jnp.bfloat16)
    g8_ref[...] = (gf * _G_SCALE).astype(_F8)
    s0 = jnp.dot(x_ref[...].astype(jnp.bfloat16),
                 w0_ref[...].astype(jnp.bfloat16),
                 preferred_element_type=jnp.float32).astype(jnp.bfloat16)
    acc = jnp.dot(gb, s0, preferred_element_type=jnp.float32)
    h = jnp.tanh(acc + b0_ref[...])
    hb = h.astype(jnp.bfloat16)
    h0_ref[...] = hb
    s1 = jnp.dot(hb, w1_ref[...].astype(jnp.bfloat16),
                 preferred_element_type=jnp.float32)
    s1_ref[...] = jnp.clip(s1 * _S_SCALE, -_F8_MAX, _F8_MAX).astype(_F8)


def _stage2(g, x, w0b, b0, w1b, tm):
    n, k = g.shape
    din = x.shape[1]
    d = w0b.shape[1]
    return pl.pallas_call(
        _layer0_kernel,
        out_shape=(jax.ShapeDtypeStruct((n, d), jnp.bfloat16),
                   jax.ShapeDtypeStruct((n, d), _F8),
                   jax.ShapeDtypeStruct((n, k), _F8)),
        grid_spec=pltpu.PrefetchScalarGridSpec(
            num_scalar_prefetch=0,
            grid=(n // tm,),
            in_specs=[
                pl.BlockSpec((tm, k), lambda i: (i, 0)),
                pl.BlockSpec((k, din), lambda i: (0, 0)),
                pl.BlockSpec((din, d), lambda i: (0, 0)),
                pl.BlockSpec((1, d), lambda i: (0, 0)),
                pl.BlockSpec((d, d), lambda i: (0, 0)),
            ],
            out_specs=(pl.BlockSpec((tm, d), lambda i: (i, 0)),
                       pl.BlockSpec((tm, d), lambda i: (i, 0)),
                       pl.BlockSpec((tm, k), lambda i: (i, 0))),
        ),
        compiler_params=_params(("parallel",)),
    )(g, x, w0b, b0, w1b)


def _layer1_kernel(alpha_ref, g_ref, s1_ref, b1_ref, h0_ref, x_ref, rw_ref,
                   gamma_ref, nb_ref, o_ref, *, eps):
    acc = jnp.dot(g_ref[...], s1_ref[...], preferred_element_type=jnp.float32)
    acc = acc * (1.0 / (_G_SCALE * _S_SCALE))
    h1 = jnp.tanh(acc + b1_ref[...])
    res = jnp.dot(x_ref[...].astype(jnp.bfloat16),
                  rw_ref[...].astype(jnp.bfloat16),
                  preferred_element_type=jnp.float32)
    c = (alpha_ref[0] * h0_ref[...].astype(jnp.float32)
         + alpha_ref[1] * h1 + res)
    d = c.shape[-1]
    mean = jnp.mean(c, axis=-1, keepdims=True)
    cc = c - mean
    var = jnp.sum(cc * cc, axis=-1, keepdims=True) / (d - 1)
    denom = jnp.sqrt(var) + eps
    inv = pl.reciprocal(denom, approx=True)
    inv = inv * (2.0 - denom * inv)
    o_ref[...] = (gamma_ref[...] * (cc * inv) + nb_ref[...]).astype(o_ref.dtype)


def _stage3(g8, s1, b1, h0, x, rw, alpha2, gamma, nbias, tm, eps):
    n, k = g8.shape
    d = s1.shape[1]
    din = x.shape[1]
    row = lambda i: (i, 0)
    const = lambda i: (0, 0)
    return pl.pallas_call(
        functools.partial(_layer1_kernel, eps=eps),
        out_shape=jax.ShapeDtypeStruct((n, d), x.dtype),
        grid_spec=pltpu.PrefetchScalarGridSpec(
            num_scalar_prefetch=0,
            grid=(n // tm,),
            in_specs=[
                pl.BlockSpec(memory_space=pltpu.MemorySpace.SMEM),
                pl.BlockSpec((tm, k), row),
                pl.BlockSpec((k, d), const),
                pl.BlockSpec((1, d), const),
                pl.BlockSpec((tm, d), row),
                pl.BlockSpec((tm, din), row),
                pl.BlockSpec((din, d), const),
                pl.BlockSpec((tm, d), row),
                pl.BlockSpec((tm, d), row),
            ],
            out_specs=pl.BlockSpec((tm, d), row),
        ),
        compiler_params=_params(("parallel",)),
    )(alpha2, g8, s1, b1, h0, x, rw, gamma, nbias)


def kernel(g, x, w0, b0, w1, b1, alpha, res_weight, gamma, norm_bias):
    n = g.shape[0]
    d = w0.shape[1]
    eps = 1e-10
    tm2 = _row_tile(n, 1024)
    tm3 = _row_tile(n, 1024)

    b0r = b0.reshape(1, d)
    b1r = b1.reshape(1, d)
    alpha2 = alpha.reshape(-1).astype(jnp.float32)

    h0, s1, g8 = _stage2(g, x, w0, b0r, w1, tm2)
    return _stage3(g8, s1, b1r, h0, x, res_weight,
                   alpha2, gamma, norm_bias, tm3, eps)
